# baseline (speedup 1.0000x reference)
import jax
import jax.numpy as jnp
from jax.experimental import pallas as pl
from jax.experimental.pallas import tpu as pltpu

HIDDEN = 768
NUM_PROPS = 32
NUM_EXPERTS = 8
BATCH = 2048
OUT_DIM = NUM_PROPS * 2


def _dot(a, b):
    return jax.lax.dot_general(
        a.astype(jnp.bfloat16), b.astype(jnp.bfloat16), (((1,), (0,)), ((), ())),
        preferred_element_type=jnp.float32)


def _router_body(mm_ref, qf_ref, rW1_ref, rb1_ref, rW2_ref, rb2_ref, comb_ref):
    col = jax.lax.broadcasted_iota(jnp.int32, (BATCH, NUM_EXPERTS), 1)
    h = _dot(mm_ref[...], rW1_ref[:HIDDEN]) + _dot(qf_ref[...], rW1_ref[HIDDEN:])
    h = jnp.maximum(h + rb1_ref[...], 0.0)
    logits = _dot(h, rW2_ref[...]) + rb2_ref[...]
    m = jnp.max(logits, axis=-1, keepdims=True)
    ex = jnp.exp(logits - m)
    p = ex / jnp.sum(ex, axis=-1, keepdims=True)
    w1 = jnp.max(p, axis=-1, keepdims=True)
    c1 = jnp.min(jnp.where(p >= w1, col, NUM_EXPERTS), axis=-1, keepdims=True)
    oh1 = col == c1
    pm = jnp.where(oh1, -jnp.inf, p)
    w2 = jnp.max(pm, axis=-1, keepdims=True)
    c2 = jnp.min(jnp.where(pm >= w2, col, NUM_EXPERTS), axis=-1, keepdims=True)
    oh2 = col == c2
    denom = w1 + w2 + 1e-6
    comb_ref[...] = (jnp.where(oh1, w1, 0.0) + jnp.where(oh2, w2, 0.0)) / denom


def _expert_body(mm_ref, comb_ref, eW1_ref, eb1_ref, eW2_ref, eb2_ref,
                 out_ref, xbf_ref, acc_ref):
    e = pl.program_id(0)
    col = jax.lax.broadcasted_iota(jnp.int32, (BATCH, NUM_EXPERTS), 1)

    @pl.when(e == 0)
    def _prep():
        xbf_ref[...] = mm_ref[...].astype(jnp.bfloat16)

    he = jnp.maximum(
        _dot(xbf_ref[...], eW1_ref[0]).astype(jnp.bfloat16)
        + eb1_ref[0].astype(jnp.bfloat16), 0.0)
    o = _dot(he, eW2_ref[0]) + eb2_ref[0]
    w_col = jnp.sum(jnp.where(col == e, comb_ref[...], 0.0), axis=-1, keepdims=True)
    contrib = w_col * o

    @pl.when(e == 0)
    def _init():
        acc_ref[...] = contrib

    @pl.when(e > 0)
    def _acc():
        acc_ref[...] += contrib

    @pl.when(e == NUM_EXPERTS - 1)
    def _fin():
        out_ref[...] = jax.nn.sigmoid(acc_ref[...])


@jax.jit
def kernel(multimodal_feat, query_feat, rW1, rb1, rW2, rb2, eW1, eb1, eW2, eb2):
    c2 = lambda: (0, 0)
    comb = pl.pallas_call(
        _router_body,
        in_specs=[
            pl.BlockSpec((BATCH, HIDDEN), lambda: (0, 0)),
            pl.BlockSpec((BATCH, HIDDEN), lambda: (0, 0)),
            pl.BlockSpec((2 * HIDDEN, HIDDEN), lambda: (0, 0)),
            pl.BlockSpec((1, HIDDEN), lambda: (0, 0)),
            pl.BlockSpec((HIDDEN, NUM_EXPERTS), lambda: (0, 0)),
            pl.BlockSpec((1, NUM_EXPERTS), lambda: (0, 0)),
        ],
        out_specs=pl.BlockSpec((BATCH, NUM_EXPERTS), lambda: (0, 0)),
        out_shape=jax.ShapeDtypeStruct((BATCH, NUM_EXPERTS), jnp.float32),
    )(multimodal_feat, query_feat, rW1, rb1.reshape(1, HIDDEN), rW2,
      rb2.reshape(1, NUM_EXPERTS))

    out = pl.pallas_call(
        _expert_body,
        grid=(NUM_EXPERTS,),
        in_specs=[
            pl.BlockSpec((BATCH, HIDDEN), lambda e: (0, 0)),
            pl.BlockSpec((BATCH, NUM_EXPERTS), lambda e: (0, 0)),
            pl.BlockSpec((1, HIDDEN, HIDDEN), lambda e: (e, 0, 0)),
            pl.BlockSpec((1, 1, HIDDEN), lambda e: (e, 0, 0)),
            pl.BlockSpec((1, HIDDEN, OUT_DIM), lambda e: (e, 0, 0)),
            pl.BlockSpec((1, 1, OUT_DIM), lambda e: (e, 0, 0)),
        ],
        out_specs=pl.BlockSpec((BATCH, OUT_DIM), lambda e: (0, 0)),
        out_shape=jax.ShapeDtypeStruct((BATCH, OUT_DIM), jnp.float32),
        scratch_shapes=[
            pltpu.VMEM((BATCH, HIDDEN), jnp.bfloat16),
            pltpu.VMEM((BATCH, OUT_DIM), jnp.float32),
        ],
        compiler_params=pltpu.CompilerParams(
            dimension_semantics=("arbitrary",),
        ),
    )(multimodal_feat, comb, eW1,
      eb1.reshape(NUM_EXPERTS, 1, HIDDEN), eW2,
      eb2.reshape(NUM_EXPERTS, 1, OUT_DIM))
    return out.reshape(BATCH * NUM_PROPS, 2)


# speedup vs baseline: 1.3595x; 1.3595x over previous
import jax
import jax.numpy as jnp
from jax.experimental import pallas as pl
from jax.experimental.pallas import tpu as pltpu

HIDDEN = 768
NUM_PROPS = 32
NUM_EXPERTS = 8
BATCH = 2048
OUT_DIM = NUM_PROPS * 2


def _dot(a, b):
    return jax.lax.dot_general(
        a.astype(jnp.bfloat16), b.astype(jnp.bfloat16), (((1,), (0,)), ((), ())),
        preferred_element_type=jnp.float32)


def _router_body(mm_ref, qf_ref, rW1_ref, rb1_ref, rW2_ref, rb2_ref, comb_ref):
    col = jax.lax.broadcasted_iota(jnp.int32, (BATCH, NUM_EXPERTS), 1)
    h = _dot(mm_ref[...], rW1_ref[:HIDDEN]) + _dot(qf_ref[...], rW1_ref[HIDDEN:])
    h = jnp.maximum(h + rb1_ref[...], 0.0)
    logits = _dot(h, rW2_ref[...]) + rb2_ref[...]
    m = jnp.max(logits, axis=-1, keepdims=True)
    ex = jnp.exp(logits - m)
    p = ex / jnp.sum(ex, axis=-1, keepdims=True)
    w1 = jnp.max(p, axis=-1, keepdims=True)
    c1 = jnp.min(jnp.where(p >= w1, col, NUM_EXPERTS), axis=-1, keepdims=True)
    oh1 = col == c1
    pm = jnp.where(oh1, -jnp.inf, p)
    w2 = jnp.max(pm, axis=-1, keepdims=True)
    c2 = jnp.min(jnp.where(pm >= w2, col, NUM_EXPERTS), axis=-1, keepdims=True)
    oh2 = col == c2
    denom = w1 + w2 + 1e-6
    comb_ref[...] = (jnp.where(oh1, w1, 0.0) + jnp.where(oh2, w2, 0.0)) / denom


def _expert_body(mm_ref, comb_ref, eW1_ref, eb1_ref, eW2_ref, eb2_ref,
                 out_ref, xbf_ref, acc_ref):
    e = pl.program_id(0)
    col = jax.lax.broadcasted_iota(jnp.int32, (BATCH, NUM_EXPERTS), 1)

    @pl.when(e == 0)
    def _prep():
        xbf_ref[...] = mm_ref[...].astype(jnp.bfloat16)

    he = jnp.maximum(
        _dot(xbf_ref[...], eW1_ref[0]).astype(jnp.bfloat16)
        + eb1_ref[0].astype(jnp.bfloat16), 0.0)
    o = _dot(he, eW2_ref[0]) + eb2_ref[0]
    w_col = jnp.sum(jnp.where(col == e, comb_ref[...], 0.0), axis=-1, keepdims=True)
    contrib = w_col * o

    @pl.when(e == 0)
    def _init():
        acc_ref[...] = contrib

    @pl.when(e > 0)
    def _acc():
        acc_ref[...] += contrib

    @pl.when(e == NUM_EXPERTS - 1)
    def _fin():
        out_ref[...] = jax.nn.sigmoid(acc_ref[...])


@jax.jit
def kernel(multimodal_feat, query_feat, rW1, rb1, rW2, rb2, eW1, eb1, eW2, eb2):
    c2 = lambda: (0, 0)
    comb = pl.pallas_call(
        _router_body,
        in_specs=[
            pl.BlockSpec((BATCH, HIDDEN), lambda: (0, 0)),
            pl.BlockSpec((BATCH, HIDDEN), lambda: (0, 0)),
            pl.BlockSpec((2 * HIDDEN, HIDDEN), lambda: (0, 0)),
            pl.BlockSpec((1, HIDDEN), lambda: (0, 0)),
            pl.BlockSpec((HIDDEN, NUM_EXPERTS), lambda: (0, 0)),
            pl.BlockSpec((1, NUM_EXPERTS), lambda: (0, 0)),
        ],
        out_specs=pl.BlockSpec((BATCH, NUM_EXPERTS), lambda: (0, 0)),
        out_shape=jax.ShapeDtypeStruct((BATCH, NUM_EXPERTS), jnp.float32),
    )(multimodal_feat, query_feat, rW1, rb1.reshape(1, HIDDEN), rW2,
      rb2.reshape(1, NUM_EXPERTS))

    out = pl.pallas_call(
        _expert_body,
        grid=(NUM_EXPERTS,),
        in_specs=[
            pl.BlockSpec((BATCH, HIDDEN), lambda e: (0, 0)),
            pl.BlockSpec((BATCH, NUM_EXPERTS), lambda e: (0, 0)),
            pl.BlockSpec((1, HIDDEN, HIDDEN), lambda e: (e, 0, 0)),
            pl.BlockSpec((1, 1, HIDDEN), lambda e: (e, 0, 0)),
            pl.BlockSpec((1, HIDDEN, OUT_DIM), lambda e: (e, 0, 0)),
            pl.BlockSpec((1, 1, OUT_DIM), lambda e: (e, 0, 0)),
        ],
        out_specs=pl.BlockSpec((BATCH, OUT_DIM), lambda e: (0, 0)),
        out_shape=jax.ShapeDtypeStruct((BATCH, OUT_DIM), jnp.float32),
        scratch_shapes=[
            pltpu.VMEM((BATCH, HIDDEN), jnp.bfloat16),
            pltpu.VMEM((BATCH, OUT_DIM), jnp.float32),
        ],
        compiler_params=pltpu.CompilerParams(
            dimension_semantics=("arbitrary",),
        ),
    )(multimodal_feat, comb, eW1,
      eb1.reshape(NUM_EXPERTS, 1, HIDDEN), eW2,
      eb2.reshape(NUM_EXPERTS, 1, OUT_DIM))
    n = BATCH * NUM_PROPS
    ev = out[:, 0::2].reshape(n, 1)
    od = out[:, 1::2].reshape(n, 1)
    return jnp.concatenate([ev, od], axis=1)
